# baseline (speedup 1.0000x reference)
import jax
import jax.numpy as jnp
from jax.experimental import pallas as pl


def _add_kernel(x_ref, t_ref, o_ref):
    o_ref[...] = x_ref[...] + t_ref[...]


def kernel(x, freqs, table):
    Nt, Nf, D = x.shape
    TB = 4
    return pl.pallas_call(
        _add_kernel,
        grid=(Nt // TB,),
        in_specs=[
            pl.BlockSpec((TB, Nf, D), lambda i: (i, 0, 0)),
            pl.BlockSpec((1, Nf, D), lambda i: (0, 0, 0)),
        ],
        out_specs=pl.BlockSpec((TB, Nf, D), lambda i: (i, 0, 0)),
        out_shape=jax.ShapeDtypeStruct((Nt, Nf, D), x.dtype),
    )(x, table[None, :, :])
